# baseline (speedup 1.0000x reference)
import functools
import math

import jax
import jax.numpy as jnp
from jax import lax
from jax.experimental import pallas as pl
from jax.experimental.pallas import tpu as pltpu
from jax.experimental.pallas import tpu_sc as plsc

N = 10000
E = 320000
NFEAT = 128
NH = 64
NCLASS = 40
NLAYERS = 2
MOMENT = 3
LAMDA = 0.5
ALPHA = 0.1

NC = 2
NS = 16
L = 16
NW = NC * NS

BN = 1000
EPW = E // NW
EB = 20000
PACK = 16384

_sc_mesh = plsc.VectorSubcoreMesh(core_axis_name="c", subcore_axis_name="s")


def _wid():
    return lax.axis_index("s") * NC + lax.axis_index("c")


def _deg_body(edges_hbm, degp_hbm, packed_hbm, sbuf, dbuf, acc):
    wid = _wid()
    z16 = jnp.zeros((L,), jnp.float32)

    def zbody(i, c):
        acc[pl.ds(i * L, L)] = z16
        return c

    lax.fori_loop(0, N // L, zbody, 0)
    pltpu.sync_copy(edges_hbm.at[pl.ds(wid * EPW, EPW)], sbuf)
    pltpu.sync_copy(edges_hbm.at[pl.ds(E + wid * EPW, EPW)], dbuf)
    ones = jnp.ones((L,), jnp.float32)

    @plsc.parallel_loop(0, EPW, step=L)
    def _(i):
        s = sbuf[pl.ds(i, L)]
        d = dbuf[pl.ds(i, L)]
        plsc.addupdate_scatter(acc, [s], ones)
        plsc.addupdate_scatter(acc, [d], ones)
        sbuf[pl.ds(i, L)] = d * PACK + s

    pltpu.sync_copy(sbuf, packed_hbm.at[pl.ds(wid * EPW, EPW)])
    pltpu.sync_copy(acc, degp_hbm.at[pl.ds(wid * N, N)])


_deg_kernel = functools.partial(
    pl.kernel,
    out_type=[
        jax.ShapeDtypeStruct((NW * N,), jnp.float32),
        jax.ShapeDtypeStruct((E,), jnp.int32),
    ],
    mesh=_sc_mesh,
    compiler_params=pltpu.CompilerParams(needs_layout_passes=False),
    scratch_types=[
        pltpu.VMEM((EPW,), jnp.int32),
        pltpu.VMEM((EPW,), jnp.int32),
        pltpu.VMEM((N,), jnp.float32),
    ],
)(_deg_body)


def _mom_body(hb_hbm, tb_hbm, packed_hbm, out_hbm, eb0, eb1, hcb, tcb,
              acc, sem0, sem1):
    wid = _wid()
    first_copy = pltpu.async_copy(packed_hbm.at[pl.ds(0, EB)], eb0, sem0)
    pltpu.sync_copy(hb_hbm.at[pl.ds(wid * N, N)], hcb)
    pltpu.sync_copy(tb_hbm.at[pl.ds(wid * N, N)], tcb)
    z16 = jnp.zeros((L,), jnp.float32)

    @plsc.parallel_loop(0, 6 * N, step=L)
    def _(i):
        acc[pl.ds(i, L)] = z16

    def make_body(ebuf):
        def body(i):
            p = ebuf[pl.ds(i, L)]
            s = jnp.bitwise_and(p, PACK - 1)
            d = jnp.right_shift(p, 14)
            tw = plsc.load_gather(tcb, [s])
            a0, a1 = plsc.unpack(plsc.bitcast(tw, jnp.bfloat16),
                                 format=plsc.PackFormat.INTERLEAVED)
            gw = plsc.load_gather(hcb, [s])
            g0, g1 = plsc.unpack(plsc.bitcast(gw, jnp.bfloat16),
                                 format=plsc.PackFormat.INTERLEAVED)
            plsc.addupdate_scatter(acc, [d], a0)
            plsc.addupdate_scatter(acc, [d + N], a1)
            a0 = a0 * g0
            a1 = a1 * g1
            plsc.addupdate_scatter(acc, [d + 2 * N], a0)
            plsc.addupdate_scatter(acc, [d + 3 * N], a1)
            a0 = a0 * g0
            a1 = a1 * g1
            plsc.addupdate_scatter(acc, [d + 4 * N], a0)
            plsc.addupdate_scatter(acc, [d + 5 * N], a1)

        return body

    bufs = (eb0, eb1)
    sems = (sem0, sem1)
    nblk = E // EB
    copies = [None, None]
    copies[0] = first_copy
    for blk in range(nblk):
        b = blk % 2
        copies[b].wait()
        if blk + 1 < nblk:
            copies[1 - b] = pltpu.async_copy(
                packed_hbm.at[pl.ds((blk + 1) * EB, EB)], bufs[1 - b],
                sems[1 - b])
        plsc.parallel_loop(0, EB, step=L, unroll=16)(make_body(bufs[b]))

    for pw in range(3):
        pltpu.sync_copy(acc.at[pl.ds((2 * pw) * N, N)],
                        out_hbm.at[pl.ds((pw * NH + wid) * N, N)])
        pltpu.sync_copy(acc.at[pl.ds((2 * pw + 1) * N, N)],
                        out_hbm.at[pl.ds((pw * NH + wid + 32) * N, N)])


_mom_kernel = functools.partial(
    pl.kernel,
    out_type=jax.ShapeDtypeStruct((3 * NH * N,), jnp.float32),
    mesh=_sc_mesh,
    compiler_params=pltpu.CompilerParams(needs_layout_passes=False),
    scratch_types=[
        pltpu.VMEM((EB,), jnp.int32),
        pltpu.VMEM((EB,), jnp.int32),
        pltpu.VMEM((N,), jnp.int32),
        pltpu.VMEM((N,), jnp.int32),
        pltpu.VMEM((6 * N,), jnp.float32),
        pltpu.SemaphoreType.DMA,
        pltpu.SemaphoreType.DMA,
    ],
)(_mom_body)


def _pack_h(h):
    lo = lax.bitcast_convert_type(h[:NH // 2, :].astype(jnp.bfloat16),
                                  jnp.uint16).astype(jnp.uint32)
    hi = lax.bitcast_convert_type(h[NH // 2:, :].astype(jnp.bfloat16),
                                  jnp.uint16).astype(jnp.uint32)
    return lax.bitcast_convert_type(jnp.left_shift(hi, 16) | lo, jnp.int32)


def _pre_body(x_ref, w_ref, b_ref, degp_ref, hT_ref, hb_ref, tb_ref, rs_ref):
    deg = jnp.sum(degp_ref[...], axis=0, keepdims=True)
    rs = lax.rsqrt(jnp.clip(deg, 1.0, None))
    rs_ref[...] = rs
    z = lax.dot_general(w_ref[...], x_ref[...], (((0,), (1,)), ((), ())),
                        preferred_element_type=jnp.float32)
    h = jnp.maximum(z + b_ref[...], 0.0)
    hT_ref[...] = h
    hb_ref[...] = _pack_h(h)
    tb_ref[...] = _pack_h(rs * h)


def _pre_kernel(x, fc0_W, fc0_b, degp):
    return pl.pallas_call(
        _pre_body,
        out_shape=[
            jax.ShapeDtypeStruct((NH, N), jnp.float32),
            jax.ShapeDtypeStruct((NH // 2, N), jnp.int32),
            jax.ShapeDtypeStruct((NH // 2, N), jnp.int32),
            jax.ShapeDtypeStruct((1, N), jnp.float32),
        ],
    )(x, fc0_W, fc0_b.reshape(NH, 1), degp)


def _layer_body(theta, last, momT_ref, rs_ref, hT_ref, h0T_ref, wa_ref,
                wt_ref, fwt_ref, fb_ref, out_ref, hb_ref, tb_ref):
    rs = rs_ref[...]
    momT = momT_ref[...] * rs
    mu = momT[0:NH, :]
    s2 = momT[NH:2 * NH, :]
    s3 = momT[2 * NH:3 * NH, :]
    sigma = jnp.sqrt(jnp.clip(s2 - mu * mu, 1e-6, None))
    m3 = jnp.sign(s3) * jnp.power(jnp.abs(s3) + 1e-6, 1.0 / 3.0)
    hT = hT_ref[...]
    wa = wa_ref[...]
    waQ = wa[:, :NH]
    waK = wa[:, NH:]
    qT = jnp.dot(waQ, hT, preferred_element_type=jnp.float32)
    k0 = jnp.dot(waK[0:1, :], mu, preferred_element_type=jnp.float32)
    k1 = jnp.dot(waK[1:2, :], sigma, preferred_element_type=jnp.float32)
    k2 = jnp.dot(waK[2:3, :], m3, preferred_element_type=jnp.float32)
    a = jnp.tanh(qT + jnp.concatenate([k0, k1, k2], axis=0))
    a = a - jnp.max(a, axis=0, keepdims=True)
    ea = jnp.exp(a)
    attn = ea / jnp.sum(ea, axis=0, keepdims=True)
    aggT = attn[0:1, :] * mu + attn[1:2, :] * sigma + attn[2:3, :] * m3
    supportT = (1.0 - ALPHA) * aggT + ALPHA * h0T_ref[...]
    zT = jnp.dot(wt_ref[...], supportT, preferred_element_type=jnp.float32)
    h = jnp.maximum(theta * zT + (1.0 - theta) * supportT, 0.0)
    if last:
        lg = jnp.dot(fwt_ref[...], h, preferred_element_type=jnp.float32)
        lg = lg + fb_ref[...]
        lg = lg - jnp.max(lg, axis=0, keepdims=True)
        out_ref[...] = lg - jnp.log(jnp.sum(jnp.exp(lg), axis=0,
                                            keepdims=True))
        hb_ref[...] = jnp.zeros_like(hb_ref)
        tb_ref[...] = jnp.zeros_like(tb_ref)
    else:
        out_ref[...] = h
        hb_ref[...] = _pack_h(h)
        tb_ref[...] = _pack_h(rs * h)


def _layer_kernel(l, momT, rs, hT, h0T, wa, wt, fwt, fb):
    theta = math.log(LAMDA / (l + 1) + 1.0)
    last = l == NLAYERS - 1
    odim = NCLASS if last else NH
    return pl.pallas_call(
        functools.partial(_layer_body, theta, last),
        out_shape=[
            jax.ShapeDtypeStruct((odim, N), jnp.float32),
            jax.ShapeDtypeStruct((1, 1) if last else (NH // 2, N), jnp.int32),
            jax.ShapeDtypeStruct((1, 1) if last else (NH // 2, N), jnp.int32),
        ],
    )(momT, rs, hT, h0T, wa, wt, fwt, fb)


def kernel(x, edge_index, fc0_W, fc0_b, conv_weight, conv_watt, fc1_W, fc1_b):
    degp_flat, packed = _deg_kernel(edge_index.reshape(2 * E))
    degp = degp_flat.reshape(NW, N)
    hT, hb, tb, rs = _pre_kernel(x, fc0_W, fc0_b, degp)
    h0T = hT
    fwt = fc1_W.T
    fb = fc1_b.reshape(NCLASS, 1)
    for l in range(NLAYERS):
        momT = _mom_kernel(hb.reshape(NH // 2 * N), tb.reshape(NH // 2 * N),
                           packed)
        hT, hb, tb = _layer_kernel(l, momT.reshape(3 * NH, N), rs, hT, h0T,
                                   conv_watt[l], conv_weight[l].T, fwt, fb)
    return hT.T


# speedup vs baseline: 1.0201x; 1.0201x over previous
import functools
import math

import jax
import jax.numpy as jnp
from jax import lax
from jax.experimental import pallas as pl
from jax.experimental.pallas import tpu as pltpu
from jax.experimental.pallas import tpu_sc as plsc

N = 10000
E = 320000
NFEAT = 128
NH = 64
NCLASS = 40
NLAYERS = 2
MOMENT = 3
LAMDA = 0.5
ALPHA = 0.1

NC = 2
NS = 16
L = 16
NW = NC * NS

BN = 1000
EPW = E // NW
EB = 20000
PACK = 16384

_sc_mesh = plsc.VectorSubcoreMesh(core_axis_name="c", subcore_axis_name="s")


def _wid():
    return lax.axis_index("s") * NC + lax.axis_index("c")


def _deg_body(edges_hbm, degp_hbm, packed_hbm, sbuf, dbuf, acc):
    wid = _wid()
    z16 = jnp.zeros((L,), jnp.float32)

    def zbody(i, c):
        acc[pl.ds(i * L, L)] = z16
        return c

    lax.fori_loop(0, N // L, zbody, 0)
    pltpu.sync_copy(edges_hbm.at[pl.ds(wid * EPW, EPW)], sbuf)
    pltpu.sync_copy(edges_hbm.at[pl.ds(E + wid * EPW, EPW)], dbuf)
    ones = jnp.ones((L,), jnp.float32)

    @plsc.parallel_loop(0, EPW, step=L)
    def _(i):
        s = sbuf[pl.ds(i, L)]
        d = dbuf[pl.ds(i, L)]
        plsc.addupdate_scatter(acc, [s], ones)
        plsc.addupdate_scatter(acc, [d], ones)
        sbuf[pl.ds(i, L)] = d * PACK + s

    pltpu.sync_copy(sbuf, packed_hbm.at[pl.ds(wid * EPW, EPW)])
    pltpu.sync_copy(acc, degp_hbm.at[pl.ds(wid * N, N)])


_deg_kernel = functools.partial(
    pl.kernel,
    out_type=[
        jax.ShapeDtypeStruct((NW * N,), jnp.float32),
        jax.ShapeDtypeStruct((E,), jnp.int32),
    ],
    mesh=_sc_mesh,
    compiler_params=pltpu.CompilerParams(needs_layout_passes=False),
    scratch_types=[
        pltpu.VMEM((EPW,), jnp.int32),
        pltpu.VMEM((EPW,), jnp.int32),
        pltpu.VMEM((N,), jnp.float32),
    ],
)(_deg_body)


def _mom_body(hb_hbm, tb_hbm, packed_hbm, out_hbm, eb0, eb1, hcb, tcb,
              acc, sem0, sem1):
    wid = _wid()
    first_copy = pltpu.async_copy(packed_hbm.at[pl.ds(0, EB)], eb0, sem0)
    pltpu.sync_copy(hb_hbm.at[wid], hcb)
    pltpu.sync_copy(tb_hbm.at[wid], tcb)
    z16 = jnp.zeros((L,), jnp.float32)

    @plsc.parallel_loop(0, 6 * N, step=L)
    def _(i):
        acc[pl.ds(i, L)] = z16

    def make_body(ebuf):
        def body(i):
            p = ebuf[pl.ds(i, L)]
            s = jnp.bitwise_and(p, PACK - 1)
            d = jnp.right_shift(p, 14)
            tw = plsc.load_gather(tcb, [s])
            a0, a1 = plsc.unpack(plsc.bitcast(tw, jnp.bfloat16),
                                 format=plsc.PackFormat.INTERLEAVED)
            gw = plsc.load_gather(hcb, [s])
            g0, g1 = plsc.unpack(plsc.bitcast(gw, jnp.bfloat16),
                                 format=plsc.PackFormat.INTERLEAVED)
            plsc.addupdate_scatter(acc, [d], a0)
            plsc.addupdate_scatter(acc, [d + N], a1)
            a0 = a0 * g0
            a1 = a1 * g1
            plsc.addupdate_scatter(acc, [d + 2 * N], a0)
            plsc.addupdate_scatter(acc, [d + 3 * N], a1)
            a0 = a0 * g0
            a1 = a1 * g1
            plsc.addupdate_scatter(acc, [d + 4 * N], a0)
            plsc.addupdate_scatter(acc, [d + 5 * N], a1)

        return body

    bufs = (eb0, eb1)
    sems = (sem0, sem1)
    nblk = E // EB
    copies = [None, None]
    copies[0] = first_copy
    for blk in range(nblk):
        b = blk % 2
        copies[b].wait()
        if blk + 1 < nblk:
            copies[1 - b] = pltpu.async_copy(
                packed_hbm.at[pl.ds((blk + 1) * EB, EB)], bufs[1 - b],
                sems[1 - b])
        plsc.parallel_loop(0, EB, step=L, unroll=16)(make_body(bufs[b]))

    for pw in range(3):
        pltpu.sync_copy(acc.at[pl.ds((2 * pw) * N, N)],
                        out_hbm.at[pl.ds((pw * NH + wid) * N, N)])
        pltpu.sync_copy(acc.at[pl.ds((2 * pw + 1) * N, N)],
                        out_hbm.at[pl.ds((pw * NH + wid + 32) * N, N)])


_mom_kernel = functools.partial(
    pl.kernel,
    out_type=jax.ShapeDtypeStruct((3 * NH * N,), jnp.float32),
    mesh=_sc_mesh,
    compiler_params=pltpu.CompilerParams(needs_layout_passes=False),
    scratch_types=[
        pltpu.VMEM((EB,), jnp.int32),
        pltpu.VMEM((EB,), jnp.int32),
        pltpu.VMEM((N,), jnp.int32),
        pltpu.VMEM((N,), jnp.int32),
        pltpu.VMEM((6 * N,), jnp.float32),
        pltpu.SemaphoreType.DMA,
        pltpu.SemaphoreType.DMA,
    ],
)(_mom_body)


def _pack_h(h):
    lo = lax.bitcast_convert_type(h[:NH // 2, :].astype(jnp.bfloat16),
                                  jnp.uint16).astype(jnp.uint32)
    hi = lax.bitcast_convert_type(h[NH // 2:, :].astype(jnp.bfloat16),
                                  jnp.uint16).astype(jnp.uint32)
    return lax.bitcast_convert_type(jnp.left_shift(hi, 16) | lo, jnp.int32)


def _pre_body(x_ref, w_ref, b_ref, degp_ref, hT_ref, hb_ref, tb_ref, rs_ref):
    deg = jnp.sum(degp_ref[...], axis=0, keepdims=True)
    rs = lax.rsqrt(jnp.clip(deg, 1.0, None))
    rs_ref[...] = rs
    z = lax.dot_general(w_ref[...], x_ref[...], (((0,), (1,)), ((), ())),
                        preferred_element_type=jnp.float32)
    h = jnp.maximum(z + b_ref[...], 0.0)
    hT_ref[...] = h
    hb_ref[...] = _pack_h(h)
    tb_ref[...] = _pack_h(rs * h)


def _pre_kernel(x, fc0_W, fc0_b, degp):
    return pl.pallas_call(
        _pre_body,
        out_shape=[
            jax.ShapeDtypeStruct((NH, N), jnp.float32),
            jax.ShapeDtypeStruct((NH // 2, N), jnp.int32),
            jax.ShapeDtypeStruct((NH // 2, N), jnp.int32),
            jax.ShapeDtypeStruct((1, N), jnp.float32),
        ],
    )(x, fc0_W, fc0_b.reshape(NH, 1), degp)


def _layer_body(theta, last, momT_ref, rs_ref, hT_ref, h0T_ref, wa_ref,
                wt_ref, fwt_ref, fb_ref, out_ref, hb_ref, tb_ref):
    rs = rs_ref[...]
    momT = momT_ref[...] * rs
    mu = momT[0:NH, :]
    s2 = momT[NH:2 * NH, :]
    s3 = momT[2 * NH:3 * NH, :]
    sigma = jnp.sqrt(jnp.clip(s2 - mu * mu, 1e-6, None))
    m3 = jnp.sign(s3) * jnp.power(jnp.abs(s3) + 1e-6, 1.0 / 3.0)
    hT = hT_ref[...]
    wa = wa_ref[...]
    waQ = wa[:, :NH]
    waK = wa[:, NH:]
    qT = jnp.dot(waQ, hT, preferred_element_type=jnp.float32)
    k0 = jnp.dot(waK[0:1, :], mu, preferred_element_type=jnp.float32)
    k1 = jnp.dot(waK[1:2, :], sigma, preferred_element_type=jnp.float32)
    k2 = jnp.dot(waK[2:3, :], m3, preferred_element_type=jnp.float32)
    a = jnp.tanh(qT + jnp.concatenate([k0, k1, k2], axis=0))
    a = a - jnp.max(a, axis=0, keepdims=True)
    ea = jnp.exp(a)
    attn = ea / jnp.sum(ea, axis=0, keepdims=True)
    aggT = attn[0:1, :] * mu + attn[1:2, :] * sigma + attn[2:3, :] * m3
    supportT = (1.0 - ALPHA) * aggT + ALPHA * h0T_ref[...]
    zT = jnp.dot(wt_ref[...], supportT, preferred_element_type=jnp.float32)
    h = jnp.maximum(theta * zT + (1.0 - theta) * supportT, 0.0)
    if last:
        lg = jnp.dot(fwt_ref[...], h, preferred_element_type=jnp.float32)
        lg = lg + fb_ref[...]
        lg = lg - jnp.max(lg, axis=0, keepdims=True)
        out_ref[...] = lg - jnp.log(jnp.sum(jnp.exp(lg), axis=0,
                                            keepdims=True))
        hb_ref[...] = jnp.zeros_like(hb_ref)
        tb_ref[...] = jnp.zeros_like(tb_ref)
    else:
        out_ref[...] = h
        hb_ref[...] = _pack_h(h)
        tb_ref[...] = _pack_h(rs * h)


def _layer_kernel(l, momT, rs, hT, h0T, wa, wt, fwt, fb):
    theta = math.log(LAMDA / (l + 1) + 1.0)
    last = l == NLAYERS - 1
    odim = NCLASS if last else NH
    return pl.pallas_call(
        functools.partial(_layer_body, theta, last),
        out_shape=[
            jax.ShapeDtypeStruct((odim, N), jnp.float32),
            jax.ShapeDtypeStruct((1, 1) if last else (NH // 2, N), jnp.int32),
            jax.ShapeDtypeStruct((1, 1) if last else (NH // 2, N), jnp.int32),
        ],
    )(momT, rs, hT, h0T, wa, wt, fwt, fb)


def kernel(x, edge_index, fc0_W, fc0_b, conv_weight, conv_watt, fc1_W, fc1_b):
    degp_flat, packed = _deg_kernel(edge_index.reshape(2 * E))
    degp = degp_flat.reshape(NW, N)
    hT, hb, tb, rs = _pre_kernel(x, fc0_W, fc0_b, degp)
    h0T = hT
    fwt = fc1_W.T
    fb = fc1_b.reshape(NCLASS, 1)
    for l in range(NLAYERS):
        momT = _mom_kernel(hb, tb, packed)
        hT, hb, tb = _layer_kernel(l, momT.reshape(3 * NH, N), rs, hT, h0T,
                                   conv_watt[l], conv_weight[l].T, fwt, fb)
    return hT.T
